# baseline (speedup 1.0000x reference)
import functools

import jax
import jax.numpy as jnp
from jax import lax
from jax.experimental import pallas as pl
from jax.experimental.pallas import tpu as pltpu
from jax.experimental.pallas import tpu_sc as plsc

MAX_NORM = 0.175

NUM_ROWS = 150000
NPAD = 150528
NCOL = 1176
B = 16384 * 200
IDX_ROWS = B // 128
NW = 32
ROWS_PER_W = IDX_ROWS // NW
K = 8
D = 8
PAT_N = K * 128 * 6 // 16


def _prep_body(x_ref, y_ref, z_ref, o_ref):
    x = x_ref[...]
    y = y_ref[...]
    z = z_ref[...]
    n = jnp.sqrt(x * x + y * y + z * z)
    scale = jnp.where(n > MAX_NORM, MAX_NORM / jnp.maximum(n, 1e-7), 1.0)
    xs = x * scale
    ys = y * scale
    zs = z * scale
    o_ref[0] = -zs
    o_ref[1] = ys
    o_ref[2] = zs
    o_ref[3] = -xs
    o_ref[4] = -ys
    o_ref[5] = xs
    o_ref[6] = jnp.zeros_like(xs)
    o_ref[7] = jnp.zeros_like(xs)


_prep = pl.pallas_call(
    _prep_body,
    out_shape=jax.ShapeDtypeStruct((D, NCOL, 128), jnp.float32),
)


def _gather_body(t_hbm, idx_hbm, out_hbm, idx_v, rows_v, compact_v, pat0_v,
                 pat1_v, sem):
    c = lax.axis_index("c")
    s = lax.axis_index("s")
    wid = s * 2 + c
    base = wid * ROWS_PER_W
    t8 = t_hbm

    def mkpat(v, carry):
        p = v * 16 + lax.iota(jnp.int32, 16)
        q = (p * 43691) >> 18
        src = q * 8 + (p - q * 6)
        pat0_v[pl.ds(v * 16, 16)] = src >> 3
        pat1_v[pl.ds(v * 16, 16)] = src & 7
        return carry

    lax.fori_loop(0, PAT_N, mkpat, 0)

    def chunk(i, carry):
        rb = base + i * K
        pltpu.sync_copy(idx_hbm.at[pl.ds(rb, K)], idx_v)
        handles = [
            pltpu.async_copy(t8.at[idx_v.at[j]],
                             rows_v.at[pl.ds(j * 128, 128)], sem)
            for j in range(K)
        ]
        for h in handles:
            h.wait()

        def comp(r, carry2):
            for m in range(8):
                o = r * 128 + m * 16
                i0 = pat0_v[pl.ds(o, 16)]
                i1 = pat1_v[pl.ds(o, 16)]
                g = plsc.load_gather(rows_v, [i0, i1])
                compact_v[r, pl.ds(m * 16, 16)] = g
            return carry2

        lax.fori_loop(0, K * 6, comp, 0)
        pltpu.sync_copy(compact_v, out_hbm.at[pl.ds(rb * 6, K * 6)])
        return carry

    lax.fori_loop(0, ROWS_PER_W // K, chunk, 0)


@functools.cache
def _make_gather():
    return pl.kernel(
        _gather_body,
        mesh=plsc.VectorSubcoreMesh(core_axis_name="c", subcore_axis_name="s"),
        compiler_params=pltpu.CompilerParams(
            use_tc_tiling_on_sc=False, needs_layout_passes=False),
        out_type=jax.ShapeDtypeStruct((IDX_ROWS * 6, 128), jnp.float32),
        scratch_types=[
            pltpu.VMEM((K, 128), jnp.int32),
            pltpu.VMEM((K * 128, D), jnp.float32),
            pltpu.VMEM((K * 6, 128), jnp.float32),
            pltpu.VMEM((PAT_N * 16,), jnp.int32),
            pltpu.VMEM((PAT_N * 16,), jnp.int32),
            pltpu.SemaphoreType.DMA,
        ],
    )


def kernel(idx, table):
    nb, nl = idx.shape
    table_p = jnp.zeros((NPAD, 3), jnp.float32).at[:NUM_ROWS].set(table)
    xc = table_p[:, 0].reshape(NCOL, 128)
    yc = table_p[:, 1].reshape(NCOL, 128)
    zc = table_p[:, 2].reshape(NCOL, 128)
    cols = _prep(xc, yc, zc)
    t8 = jnp.transpose(cols, (1, 2, 0)).reshape(NPAD, D)
    idx2d = idx.astype(jnp.int32).reshape(IDX_ROWS, 128)
    out = _make_gather()(t8, idx2d)
    return out.reshape(nb, nl, 3, 2)


# speedup vs baseline: 11.9813x; 11.9813x over previous
import functools

import jax
import jax.numpy as jnp
from jax import lax
from jax.experimental import pallas as pl
from jax.experimental.pallas import tpu as pltpu
from jax.experimental.pallas import tpu_sc as plsc

MAX_NORM = 0.175

NUM_ROWS = 150000
NPAD = 150528
NCOL = 1176
NB = 16384
NJ = 200
NT = NB // 128
NW = 32
TB = 8
UNITS = NJ * (NT // TB)
UNITS_PER_W = UNITS // NW
D = 8


def _prep_body(x_ref, y_ref, z_ref, o_ref):
    x = x_ref[...]
    y = y_ref[...]
    z = z_ref[...]
    n = jnp.sqrt(x * x + y * y + z * z)
    scale = jnp.where(n > MAX_NORM, MAX_NORM / jnp.maximum(n, 1e-7), 1.0)
    xs = x * scale
    ys = y * scale
    zs = z * scale
    o_ref[0] = -zs
    o_ref[1] = ys
    o_ref[2] = zs
    o_ref[3] = -xs
    o_ref[4] = -ys
    o_ref[5] = xs
    o_ref[6] = jnp.zeros_like(xs)
    o_ref[7] = jnp.zeros_like(xs)


_prep = pl.pallas_call(
    _prep_body,
    out_shape=jax.ShapeDtypeStruct((D, NCOL, 128), jnp.float32),
)


def _gather_body(t_hbm, idx_hbm, out_hbm, idx_v, rows_v, outbuf_v, sem):
    c = lax.axis_index("c")
    s = lax.axis_index("s")
    wid = s * 2 + c
    base = wid * UNITS_PER_W

    def unit(n, carry):
        u = base + n
        j = u >> 4
        tb = u & 15
        pltpu.sync_copy(idx_hbm.at[j, pl.ds(tb * TB, TB)], idx_v)
        handles = [
            pltpu.async_copy(t_hbm.at[idx_v.at[jj]],
                             rows_v.at[pl.ds(jj * 128, 128)], sem)
            for jj in range(TB)
        ]
        for h in handles:
            h.wait()

        def comp(m, carry2):
            i0 = m * 16 + lax.iota(jnp.int32, 16)
            tp = m >> 3
            lb = (m & 7) * 16
            for cc in range(6):
                i1 = jnp.full((16,), cc, jnp.int32)
                g = plsc.load_gather(rows_v, [i0, i1])
                outbuf_v[cc // 2, tp, cc % 2, pl.ds(lb, 16)] = g
            return carry2

        lax.fori_loop(0, TB * 8, comp, 0)
        for k in range(3):
            pltpu.sync_copy(outbuf_v.at[k],
                            out_hbm.at[j * 3 + k, pl.ds(tb * TB, TB)])
        return carry

    lax.fori_loop(0, UNITS_PER_W, unit, 0)


@functools.cache
def _make_gather():
    return pl.kernel(
        _gather_body,
        mesh=plsc.VectorSubcoreMesh(core_axis_name="c", subcore_axis_name="s"),
        compiler_params=pltpu.CompilerParams(
            use_tc_tiling_on_sc=False, needs_layout_passes=False),
        out_type=jax.ShapeDtypeStruct((NJ * 3, NT, 2, 128), jnp.float32),
        scratch_types=[
            pltpu.VMEM((TB, 128), jnp.int32),
            pltpu.VMEM((TB * 128, D), jnp.float32),
            pltpu.VMEM((3, TB, 2, 128), jnp.float32),
            pltpu.SemaphoreType.DMA,
        ],
    )


def kernel(idx, table):
    nb, nl = idx.shape
    table_p = jnp.zeros((NPAD, 3), jnp.float32).at[:NUM_ROWS].set(table)
    xc = table_p[:, 0].reshape(NCOL, 128)
    yc = table_p[:, 1].reshape(NCOL, 128)
    zc = table_p[:, 2].reshape(NCOL, 128)
    cols = _prep(xc, yc, zc)
    t8 = jnp.transpose(cols, (1, 2, 0)).reshape(NPAD, D)
    idx_t = jnp.transpose(idx.astype(jnp.int32)).reshape(NJ, NT, 128)
    out = _make_gather()(t8, idx_t)
    a = out.reshape(NJ, 3, NT, 2, 128)
    b = jnp.transpose(a, (2, 4, 0, 1, 3))
    return b.reshape(nb, nl, 3, 2)


# speedup vs baseline: 22.4774x; 1.8760x over previous
import functools

import jax
import jax.numpy as jnp
from jax import lax
from jax.experimental import pallas as pl
from jax.experimental.pallas import tpu as pltpu
from jax.experimental.pallas import tpu_sc as plsc

MAX_NORM = 0.175

NUM_ROWS = 150000
NPAD = 150528
NCOL = 1176
NB = 16384
NJ = 200
NT = NB // 128
NW = 32
TB = 8
UNITS = NJ * (NT // TB)
UNITS_PER_W = UNITS // NW
D = 8


def _prep_body(x_ref, y_ref, z_ref, o_ref):
    x = x_ref[...]
    y = y_ref[...]
    z = z_ref[...]
    n = jnp.sqrt(x * x + y * y + z * z)
    scale = jnp.where(n > MAX_NORM, MAX_NORM / jnp.maximum(n, 1e-7), 1.0)
    xs = x * scale
    ys = y * scale
    zs = z * scale
    o_ref[0] = -zs
    o_ref[1] = ys
    o_ref[2] = zs
    o_ref[3] = -xs
    o_ref[4] = -ys
    o_ref[5] = xs
    o_ref[6] = jnp.zeros_like(xs)
    o_ref[7] = jnp.zeros_like(xs)


_prep = pl.pallas_call(
    _prep_body,
    out_shape=jax.ShapeDtypeStruct((D, NCOL, 128), jnp.float32),
)


def _gather_body(t_hbm, idx_hbm, out_hbm,
                 idx_v0, idx_v1, rows_v0, rows_v1, outbuf_v0, outbuf_v1,
                 sem_i0, sem_i1, sem_g0, sem_g1, sem_o0, sem_o1):
    c = lax.axis_index("c")
    s = lax.axis_index("s")
    wid = s * 2 + c
    base = wid * UNITS_PER_W
    nlast = UNITS_PER_W - 1
    slot = ((idx_v0, rows_v0, outbuf_v0, sem_i0, sem_g0, sem_o0),
            (idx_v1, rows_v1, outbuf_v1, sem_i1, sem_g1, sem_o1))

    def ju(n):
        u = base + jnp.minimum(n, nlast)
        return u >> 4, u & 15

    def start_idx(n, b):
        j, tb = ju(n)
        idx_v, _, _, sem_i, _, _ = slot[b]
        pltpu.async_copy(idx_hbm.at[j, pl.ds(tb * TB, TB)], idx_v, sem_i)

    def start_gathers(n, b):
        idx_v, rows_v, _, sem_i, sem_g, _ = slot[b]
        pltpu.make_async_copy(idx_hbm.at[0, pl.ds(0, TB)], idx_v,
                              sem_i).wait()
        for jj in range(TB):
            pltpu.async_copy(t_hbm.at[idx_v.at[jj]],
                             rows_v.at[pl.ds(jj * 128, 128)], sem_g)

    def wait_gathers(b):
        idx_v, rows_v, _, _, sem_g, _ = slot[b]
        for jj in range(TB):
            pltpu.make_async_copy(t_hbm.at[idx_v.at[jj]],
                                  rows_v.at[pl.ds(jj * 128, 128)],
                                  sem_g).wait()

    def drain_out(b):
        _, _, outbuf_v, _, _, sem_o = slot[b]
        for k in range(3):
            pltpu.make_async_copy(out_hbm.at[0, pl.ds(0, TB)],
                                  outbuf_v.at[k], sem_o).wait()

    def compact(b):
        _, rows_v, outbuf_v, _, _, _ = slot[b]

        def comp(m, carry2):
            i0 = m * 16 + lax.iota(jnp.int32, 16)
            tp = m >> 3
            lb = (m & 7) * 16
            for cc in range(6):
                i1 = jnp.full((16,), cc, jnp.int32)
                g = plsc.load_gather(rows_v, [i0, i1])
                outbuf_v[cc // 2, tp, cc % 2, pl.ds(lb, 16)] = g
            return carry2

        lax.fori_loop(0, TB * 8, comp, 0)

    def start_out(n, b):
        j, tb = ju(n)
        _, _, outbuf_v, _, _, sem_o = slot[b]
        for k in range(3):
            pltpu.async_copy(outbuf_v.at[k],
                             out_hbm.at[j * 3 + k, pl.ds(tb * TB, TB)],
                             sem_o)

    start_idx(0, 0)
    start_gathers(0, 0)
    start_idx(1, 1)

    def pipe(g, carry):
        for b in (0, 1):
            n = g * 2 + b
            nb = 1 - b

            @pl.when(n + 1 <= nlast)
            def _():
                start_gathers(n + 1, nb)

            wait_gathers(b)

            @pl.when(n + 2 <= nlast)
            def _():
                start_idx(n + 2, b)

            @pl.when(n >= 2)
            def _():
                drain_out(b)

            compact(b)
            start_out(n, b)
        return carry

    lax.fori_loop(0, UNITS_PER_W // 2, pipe, 0)
    drain_out(0)
    drain_out(1)


@functools.cache
def _make_gather():
    return pl.kernel(
        _gather_body,
        mesh=plsc.VectorSubcoreMesh(core_axis_name="c", subcore_axis_name="s"),
        compiler_params=pltpu.CompilerParams(
            use_tc_tiling_on_sc=False, needs_layout_passes=False),
        out_type=jax.ShapeDtypeStruct((NJ * 3, NT, 2, 128), jnp.float32),
        scratch_types=[
            pltpu.VMEM((TB, 128), jnp.int32),
            pltpu.VMEM((TB, 128), jnp.int32),
            pltpu.VMEM((TB * 128, D), jnp.float32),
            pltpu.VMEM((TB * 128, D), jnp.float32),
            pltpu.VMEM((3, TB, 2, 128), jnp.float32),
            pltpu.VMEM((3, TB, 2, 128), jnp.float32),
            pltpu.SemaphoreType.DMA,
            pltpu.SemaphoreType.DMA,
            pltpu.SemaphoreType.DMA,
            pltpu.SemaphoreType.DMA,
            pltpu.SemaphoreType.DMA,
            pltpu.SemaphoreType.DMA,
        ],
    )


def kernel(idx, table):
    nb, nl = idx.shape
    table_p = jnp.zeros((NPAD, 3), jnp.float32).at[:NUM_ROWS].set(table)
    xc = table_p[:, 0].reshape(NCOL, 128)
    yc = table_p[:, 1].reshape(NCOL, 128)
    zc = table_p[:, 2].reshape(NCOL, 128)
    cols = _prep(xc, yc, zc)
    t8 = jnp.transpose(cols, (1, 2, 0)).reshape(NPAD, D)
    idx_t = jnp.transpose(idx.astype(jnp.int32)).reshape(NJ, NT, 128)
    out = _make_gather()(t8, idx_t)
    a = out.reshape(NJ, 3, NT, 2, 128)
    b = jnp.transpose(a, (2, 4, 0, 1, 3))
    return b.reshape(nb, nl, 3, 2)


# speedup vs baseline: 22.5248x; 1.0021x over previous
import functools

import jax
import jax.numpy as jnp
from jax import lax
from jax.experimental import pallas as pl
from jax.experimental.pallas import tpu as pltpu
from jax.experimental.pallas import tpu_sc as plsc

MAX_NORM = 0.175

NUM_ROWS = 150000
NPAD = 150528
NCOL = 1176
NB = 16384
NJ = 200
NT = NB // 128
NW = 32
TB = 16
PJ = NT // TB
PJ_SHIFT = PJ.bit_length() - 1
UNITS = NJ * PJ
UNITS_PER_W = UNITS // NW
D = 8


def _prep_body(x_ref, y_ref, z_ref, o_ref):
    x = x_ref[...]
    y = y_ref[...]
    z = z_ref[...]
    n = jnp.sqrt(x * x + y * y + z * z)
    scale = jnp.where(n > MAX_NORM, MAX_NORM / jnp.maximum(n, 1e-7), 1.0)
    xs = x * scale
    ys = y * scale
    zs = z * scale
    o_ref[0] = -zs
    o_ref[1] = ys
    o_ref[2] = zs
    o_ref[3] = -xs
    o_ref[4] = -ys
    o_ref[5] = xs
    o_ref[6] = jnp.zeros_like(xs)
    o_ref[7] = jnp.zeros_like(xs)


_prep = pl.pallas_call(
    _prep_body,
    out_shape=jax.ShapeDtypeStruct((D, NCOL, 128), jnp.float32),
)


def _gather_body(t_hbm, idx_hbm, out_hbm,
                 idx_v0, idx_v1, rows_v0, rows_v1, outbuf_v0, outbuf_v1,
                 sem_i0, sem_i1, sem_g0, sem_g1, sem_o0, sem_o1):
    c = lax.axis_index("c")
    s = lax.axis_index("s")
    wid = s * 2 + c
    base = wid * UNITS_PER_W
    nlast = UNITS_PER_W - 1
    slot = ((idx_v0, rows_v0, outbuf_v0, sem_i0, sem_g0, sem_o0),
            (idx_v1, rows_v1, outbuf_v1, sem_i1, sem_g1, sem_o1))

    def ju(n):
        u = base + jnp.minimum(n, nlast)
        return u >> PJ_SHIFT, u & (PJ - 1)

    def start_idx(n, b):
        j, tb = ju(n)
        idx_v, _, _, sem_i, _, _ = slot[b]
        pltpu.async_copy(idx_hbm.at[j, pl.ds(tb * TB, TB)], idx_v, sem_i)

    def start_gathers(n, b):
        idx_v, rows_v, _, sem_i, sem_g, _ = slot[b]
        pltpu.make_async_copy(idx_hbm.at[0, pl.ds(0, TB)], idx_v,
                              sem_i).wait()
        for jj in range(TB):
            pltpu.async_copy(t_hbm.at[idx_v.at[jj]],
                             rows_v.at[pl.ds(jj * 128, 128)], sem_g)

    def wait_gathers(b):
        idx_v, rows_v, _, _, sem_g, _ = slot[b]
        for jj in range(TB):
            pltpu.make_async_copy(t_hbm.at[idx_v.at[jj]],
                                  rows_v.at[pl.ds(jj * 128, 128)],
                                  sem_g).wait()

    def drain_out(b):
        _, _, outbuf_v, _, _, sem_o = slot[b]
        for k in range(3):
            pltpu.make_async_copy(out_hbm.at[0, pl.ds(0, TB)],
                                  outbuf_v.at[k], sem_o).wait()

    def compact(b):
        _, rows_v, outbuf_v, _, _, _ = slot[b]

        def comp(m, carry2):
            i0 = m * 16 + lax.iota(jnp.int32, 16)
            tp = m >> 3
            lb = (m & 7) * 16
            for cc in range(6):
                i1 = jnp.full((16,), cc, jnp.int32)
                g = plsc.load_gather(rows_v, [i0, i1])
                outbuf_v[cc // 2, tp, cc % 2, pl.ds(lb, 16)] = g
            return carry2

        lax.fori_loop(0, TB * 8, comp, 0)

    def start_out(n, b):
        j, tb = ju(n)
        _, _, outbuf_v, _, _, sem_o = slot[b]
        for k in range(3):
            pltpu.async_copy(outbuf_v.at[k],
                             out_hbm.at[j * 3 + k, pl.ds(tb * TB, TB)],
                             sem_o)

    start_idx(0, 0)
    start_gathers(0, 0)
    start_idx(1, 1)

    def pipe(g, carry):
        for b in (0, 1):
            n = g * 2 + b
            nb = 1 - b

            @pl.when(n + 1 <= nlast)
            def _():
                start_gathers(n + 1, nb)

            wait_gathers(b)

            @pl.when(n + 2 <= nlast)
            def _():
                start_idx(n + 2, b)

            @pl.when(n >= 2)
            def _():
                drain_out(b)

            compact(b)
            start_out(n, b)
        return carry

    lax.fori_loop(0, UNITS_PER_W // 2, pipe, 0)
    drain_out(0)
    drain_out(1)


@functools.cache
def _make_gather():
    return pl.kernel(
        _gather_body,
        mesh=plsc.VectorSubcoreMesh(core_axis_name="c", subcore_axis_name="s"),
        compiler_params=pltpu.CompilerParams(
            use_tc_tiling_on_sc=False, needs_layout_passes=False),
        out_type=jax.ShapeDtypeStruct((NJ * 3, NT, 2, 128), jnp.float32),
        scratch_types=[
            pltpu.VMEM((TB, 128), jnp.int32),
            pltpu.VMEM((TB, 128), jnp.int32),
            pltpu.VMEM((TB * 128, D), jnp.float32),
            pltpu.VMEM((TB * 128, D), jnp.float32),
            pltpu.VMEM((3, TB, 2, 128), jnp.float32),
            pltpu.VMEM((3, TB, 2, 128), jnp.float32),
            pltpu.SemaphoreType.DMA,
            pltpu.SemaphoreType.DMA,
            pltpu.SemaphoreType.DMA,
            pltpu.SemaphoreType.DMA,
            pltpu.SemaphoreType.DMA,
            pltpu.SemaphoreType.DMA,
        ],
    )


def kernel(idx, table):
    nb, nl = idx.shape
    table_p = jnp.zeros((NPAD, 3), jnp.float32).at[:NUM_ROWS].set(table)
    xc = table_p[:, 0].reshape(NCOL, 128)
    yc = table_p[:, 1].reshape(NCOL, 128)
    zc = table_p[:, 2].reshape(NCOL, 128)
    cols = _prep(xc, yc, zc)
    t8 = jnp.transpose(cols, (1, 2, 0)).reshape(NPAD, D)
    idx_t = jnp.transpose(idx.astype(jnp.int32)).reshape(NJ, NT, 128)
    out = _make_gather()(t8, idx_t)
    a = out.reshape(NJ, 3, NT, 2, 128)
    b = jnp.transpose(a, (2, 4, 0, 1, 3))
    return b.reshape(nb, nl, 3, 2)
